# Initial kernel scaffold; baseline (speedup 1.0000x reference)
import jax
import jax.numpy as jnp
from jax.experimental import pallas as pl


def kernel(x, edge_index, batch, W_rel, b_rel, W_root, W_lin, b_lin):
    raise NotImplementedError("write your pallas kernel here")


# speedup vs baseline: 8.8054x; 8.8054x over previous
import functools

import jax
import jax.numpy as jnp
from jax import lax
from jax.experimental import pallas as pl
from jax.experimental.pallas import tpu as pltpu
from jax.experimental.pallas import tpu_sc as plsc

_N_NODES = 10000
_N_EDGES = 320000
_N_GRAPHS = 64
_NC = 2
_NS = 16
_ROWS = 640
_N_PAD = _ROWS * _NS
_EDGES_PER_CORE = _N_EDGES // _NC
_CHUNK = 8000
_UNROLL = 4


def _hist_body(src_hbm, dst_hbm, batch_hbm, out_hbm, batch_v, src_v, dst_v, c_v):
    cid = lax.axis_index("c")
    sid = lax.axis_index("s")
    lo = sid * _ROWS

    pltpu.sync_copy(batch_hbm, batch_v)

    def _zero(r, carry):
        for j in range(_N_GRAPHS // 16):
            c_v[r, pl.ds(j * 16, 16)] = jnp.zeros((16,), jnp.float32)
        return carry

    lax.fori_loop(0, _ROWS, _zero, 0)

    ones = jnp.ones((16,), jnp.float32)
    ebase = cid * _EDGES_PER_CORE

    def _chunk(k, carry):
        off = ebase + k * _CHUNK
        pltpu.sync_copy(src_hbm.at[pl.ds(off, _CHUNK)], src_v)
        pltpu.sync_copy(dst_hbm.at[pl.ds(off, _CHUNK)], dst_v)

        def _grp(i, c2):
            for u in range(_UNROLL):
                base = (i * _UNROLL + u) * 16
                sv = src_v[pl.ds(base, 16)]
                dv = dst_v[pl.ds(base, 16)]
                ge = plsc.load_gather(batch_v, [dv])
                m = (sv >= lo) & (sv < lo + _ROWS)
                local = jnp.where(m, sv - lo, 0)
                plsc.addupdate_scatter(c_v, [local, ge], ones, mask=m)
            return c2

        lax.fori_loop(0, _CHUNK // (16 * _UNROLL), _grp, 0)
        return carry

    lax.fori_loop(0, _EDGES_PER_CORE // _CHUNK, _chunk, 0)

    pltpu.sync_copy(c_v, out_hbm.at[cid, pl.ds(lo, _ROWS)])


@functools.cache
def _edge_hist():
    return functools.partial(
        pl.kernel,
        mesh=plsc.VectorSubcoreMesh(core_axis_name="c", subcore_axis_name="s"),
        out_type=jax.ShapeDtypeStruct((_NC, _N_PAD, _N_GRAPHS), jnp.float32),
        compiler_params=pltpu.CompilerParams(needs_layout_passes=False),
        scratch_types=[
            pltpu.VMEM((_N_NODES,), jnp.int32),
            pltpu.VMEM((_CHUNK,), jnp.int32),
            pltpu.VMEM((_CHUNK,), jnp.int32),
            pltpu.VMEM((_ROWS, _N_GRAPHS), jnp.float32),
        ],
    )(_hist_body)


def _pool_body(x_ref, cp_ref, b_ref, wrel_ref, brel_ref, wroot_ref,
               wlin_ref, blin_ref, out_ref):
    f32 = jnp.float32
    hi = lax.Precision.HIGHEST
    x = x_ref[...]
    cn = (cp_ref[0] + cp_ref[1])[:_N_NODES]
    giota = lax.broadcasted_iota(jnp.int32, (_N_NODES, _N_GRAPHS), 1)
    onehot = (b_ref[...] == giota).astype(f32)
    dn = (((0,), (0,)), ((), ()))
    m1 = lax.dot_general(cn, x, dn, precision=hi, preferred_element_type=f32)
    m2 = lax.dot_general(onehot, x, dn, precision=hi, preferred_element_type=f32)
    ncol = lax.dot_general(onehot, jnp.ones((_N_NODES, 1), f32), dn,
                           precision=hi, preferred_element_type=f32)
    dc = (((1,), (1,)), ((), ()))
    sums = (lax.dot_general(m1, wrel_ref[...], dc, precision=hi,
                            preferred_element_type=f32)
            + lax.dot_general(m2, wroot_ref[...], dc, precision=hi,
                              preferred_element_type=f32)
            + ncol * brel_ref[...])
    pooled = sums / jnp.maximum(ncol, 1.0)
    out_ref[...] = (lax.dot_general(pooled, wlin_ref[...], dc, precision=hi,
                                    preferred_element_type=f32)
                    + blin_ref[...])


def kernel(x, edge_index, batch, W_rel, b_rel, W_root, W_lin, b_lin):
    n_classes = W_lin.shape[0]
    cpart = _edge_hist()(edge_index[0], edge_index[1], batch)
    pool = pl.pallas_call(
        _pool_body,
        out_shape=jax.ShapeDtypeStruct((_N_GRAPHS, n_classes), jnp.float32),
    )
    return pool(x, cpart, batch.reshape(_N_NODES, 1), W_rel,
                b_rel.reshape(1, -1), W_root, W_lin, b_lin.reshape(1, -1))


# speedup vs baseline: 24.5279x; 2.7856x over previous
import functools

import jax
import jax.numpy as jnp
from jax import lax
from jax.experimental import pallas as pl
from jax.experimental.pallas import tpu as pltpu
from jax.experimental.pallas import tpu_sc as plsc

_N_NODES = 10000
_N_EDGES = 320000
_N_GRAPHS = 64
_NC = 2
_NS = 16
_N_PAD = 10240
_KEYS = _N_PAD * _N_GRAPHS
_SLICE = _KEYS // _NS
_EPT = _N_EDGES // (_NC * _NS)
_EPTP = 10240
_ICH = 128
_NCH = _EPTP // _ICH


def _hist_body(src_hbm, dst_hbm, batch_hbm, out_hbm,
               batch_v, src_v, dst_v, key_v, ones_v, stage_v, c_sh, sem):
    cid = lax.axis_index("c")
    sid = lax.axis_index("s")
    ebase = (cid * _NS + sid) * _EPT

    pltpu.sync_copy(batch_hbm, batch_v)
    pltpu.sync_copy(src_hbm.at[pl.ds(ebase, _EPT)], src_v.at[pl.ds(0, _EPT)])
    pltpu.sync_copy(dst_hbm.at[pl.ds(ebase, _EPT)], dst_v.at[pl.ds(0, _EPT)])

    for i in range(_EPT, _EPTP, 16):
        src_v[pl.ds(i, 16)] = jnp.full((16,), _N_PAD - 1, jnp.int32)
        dst_v[pl.ds(i, 16)] = jnp.zeros((16,), jnp.int32)
    for i in range(0, _ICH, 16):
        ones_v[pl.ds(i, 16)] = jnp.ones((16,), jnp.float32)

    def _zero(i, carry):
        stage_v[pl.ds(i * 16, 16)] = jnp.zeros((16,), jnp.float32)
        return carry

    lax.fori_loop(0, _SLICE // 16, _zero, 0, unroll=8)
    pltpu.sync_copy(stage_v, c_sh.at[pl.ds(sid * _SLICE, _SLICE)])

    def _keys(j, carry):
        for u in range(_ICH // 16):
            sv = src_v[pl.ds(j * _ICH + u * 16, 16)]
            dv = dst_v[pl.ds(j * _ICH + u * 16, 16)]
            ge = plsc.load_gather(batch_v, [dv])
            key_v[j, pl.ds(u * 16, 16)] = sv * _N_GRAPHS + ge
        return carry

    lax.fori_loop(0, _NCH, _keys, 0)

    plsc.subcore_barrier()

    def _fire(j, carry):
        pltpu.async_copy(ones_v, c_sh.at[key_v.at[j]], sem, add=True)
        return carry

    lax.fori_loop(0, _NCH, _fire, 0)

    def _drain(j, carry):
        pltpu.make_async_copy(ones_v, c_sh.at[key_v.at[0]], sem).wait()
        return carry

    lax.fori_loop(0, _NCH, _drain, 0)

    plsc.subcore_barrier()

    pltpu.sync_copy(c_sh.at[pl.ds(sid * _SLICE, _SLICE)], stage_v)
    pltpu.sync_copy(stage_v, out_hbm.at[pl.ds(cid * _KEYS + sid * _SLICE,
                                              _SLICE)])


@functools.cache
def _edge_hist():
    return functools.partial(
        pl.kernel,
        mesh=plsc.VectorSubcoreMesh(core_axis_name="c", subcore_axis_name="s"),
        out_type=jax.ShapeDtypeStruct((_NC * _KEYS,), jnp.float32),
        compiler_params=pltpu.CompilerParams(needs_layout_passes=False),
        scratch_types=[
            pltpu.VMEM((_N_NODES,), jnp.int32),
            pltpu.VMEM((_EPTP,), jnp.int32),
            pltpu.VMEM((_EPTP,), jnp.int32),
            pltpu.VMEM((_NCH, _ICH), jnp.int32),
            pltpu.VMEM((_ICH,), jnp.float32),
            pltpu.VMEM((_SLICE,), jnp.float32),
            pltpu.VMEM_SHARED((_KEYS,), jnp.float32),
            pltpu.SemaphoreType.DMA,
        ],
    )(_hist_body)


def _pool_body(x_ref, cp_ref, b_ref, wrel_ref, brel_ref, wroot_ref,
               wlin_ref, blin_ref, out_ref):
    f32 = jnp.float32
    hi = lax.Precision.HIGHEST
    x = x_ref[...]
    cn = (cp_ref[0] + cp_ref[1])[:_N_NODES]
    giota = lax.broadcasted_iota(jnp.int32, (_N_NODES, _N_GRAPHS), 1)
    onehot = (b_ref[...] == giota).astype(f32)
    dn = (((0,), (0,)), ((), ()))
    m1 = lax.dot_general(cn, x, dn, precision=hi, preferred_element_type=f32)
    m2 = lax.dot_general(onehot, x, dn, precision=hi, preferred_element_type=f32)
    ncol = lax.dot_general(onehot, jnp.ones((_N_NODES, 1), f32), dn,
                           precision=hi, preferred_element_type=f32)
    dc = (((1,), (1,)), ((), ()))
    sums = (lax.dot_general(m1, wrel_ref[...], dc, precision=hi,
                            preferred_element_type=f32)
            + lax.dot_general(m2, wroot_ref[...], dc, precision=hi,
                              preferred_element_type=f32)
            + ncol * brel_ref[...])
    pooled = sums / jnp.maximum(ncol, 1.0)
    out_ref[...] = (lax.dot_general(pooled, wlin_ref[...], dc, precision=hi,
                                    preferred_element_type=f32)
                    + blin_ref[...])


def kernel(x, edge_index, batch, W_rel, b_rel, W_root, W_lin, b_lin):
    n_classes = W_lin.shape[0]
    cpart = _edge_hist()(edge_index[0], edge_index[1], batch)
    cpart = cpart.reshape(_NC, _N_PAD, _N_GRAPHS)
    pool = pl.pallas_call(
        _pool_body,
        out_shape=jax.ShapeDtypeStruct((_N_GRAPHS, n_classes), jnp.float32),
    )
    return pool(x, cpart, batch.reshape(_N_NODES, 1), W_rel,
                b_rel.reshape(1, -1), W_root, W_lin, b_lin.reshape(1, -1))


# speedup vs baseline: 31.3185x; 1.2769x over previous
import functools

import jax
import jax.numpy as jnp
from jax import lax
from jax.experimental import pallas as pl
from jax.experimental.pallas import tpu as pltpu
from jax.experimental.pallas import tpu_sc as plsc

_N_NODES = 10000
_N_EDGES = 320000
_N_GRAPHS = 64
_NC = 2
_NS = 16
_N_PAD = 10240
_ROWS = _N_PAD // _NS
_KEYS = _N_PAD * _N_GRAPHS
_SLICE = _KEYS // _NS
_EPT = _N_EDGES // (_NC * _NS)
_EPTP = 10240
_ICH = 128
_NCH = _EPTP // _ICH


def _hist_body(eflat_hbm, batch_hbm, out_hbm,
               batch_v, src_v, dst_v, key_v, ones_v, stage_v, c_sh, sem):
    cid = lax.axis_index("c")
    sid = lax.axis_index("s")
    ebase = (cid * _NS + sid) * _EPT

    pltpu.sync_copy(batch_hbm, batch_v)
    pltpu.sync_copy(eflat_hbm.at[pl.ds(ebase, _EPT)], src_v.at[pl.ds(0, _EPT)])
    pltpu.sync_copy(eflat_hbm.at[pl.ds(_N_EDGES + ebase, _EPT)],
                    dst_v.at[pl.ds(0, _EPT)])

    for i in range(_EPT, _EPTP, 16):
        src_v[pl.ds(i, 16)] = jnp.full((16,), _N_PAD - 1, jnp.int32)
        dst_v[pl.ds(i, 16)] = jnp.zeros((16,), jnp.int32)
    for i in range(0, _ICH, 16):
        ones_v[pl.ds(i, 16)] = jnp.ones((16,), jnp.float32)

    def _zero(i, carry):
        stage_v[pl.ds(i * 16, 16)] = jnp.zeros((16,), jnp.float32)
        return carry

    lax.fori_loop(0, _SLICE // 16, _zero, 0, unroll=8)
    pltpu.sync_copy(stage_v, c_sh.at[pl.ds(sid * _SLICE, _SLICE)])

    def _keys(j, carry):
        for u in range(_ICH // 16):
            sv = src_v[pl.ds(j * _ICH + u * 16, 16)]
            dv = dst_v[pl.ds(j * _ICH + u * 16, 16)]
            ge = plsc.load_gather(batch_v, [dv])
            key_v[j, pl.ds(u * 16, 16)] = sv * _N_GRAPHS + ge
        return carry

    lax.fori_loop(0, _NCH, _keys, 0)

    plsc.subcore_barrier()

    def _fire(j, carry):
        pltpu.async_copy(ones_v, c_sh.at[key_v.at[j]], sem, add=True)
        return carry

    lax.fori_loop(0, _NCH, _fire, 0)

    def _drain(j, carry):
        pltpu.make_async_copy(ones_v, c_sh.at[key_v.at[0]], sem).wait()
        return carry

    lax.fori_loop(0, _NCH, _drain, 0)

    plsc.subcore_barrier()

    pltpu.sync_copy(c_sh.at[pl.ds(sid * _SLICE, _SLICE)], stage_v)
    pltpu.sync_copy(stage_v, out_hbm.at[pl.ds(cid * _KEYS + sid * _SLICE,
                                              _SLICE)])


@functools.cache
def _edge_hist():
    return functools.partial(
        pl.kernel,
        mesh=plsc.VectorSubcoreMesh(core_axis_name="c", subcore_axis_name="s"),
        out_type=jax.ShapeDtypeStruct((_NC * _KEYS,), jnp.float32),
        compiler_params=pltpu.CompilerParams(needs_layout_passes=False),
        scratch_types=[
            pltpu.VMEM((_N_NODES,), jnp.int32),
            pltpu.VMEM((_EPTP,), jnp.int32),
            pltpu.VMEM((_EPTP,), jnp.int32),
            pltpu.VMEM((_NCH, _ICH), jnp.int32),
            pltpu.VMEM((_ICH,), jnp.float32),
            pltpu.VMEM((_SLICE,), jnp.float32),
            pltpu.VMEM_SHARED((_KEYS,), jnp.float32),
            pltpu.SemaphoreType.DMA,
        ],
    )(_hist_body)


def _pool_body(x_ref, cp_ref, b_ref, wrel_ref, brel_ref, wroot_ref,
               wlin_ref, blin_ref, out_ref):
    f32 = jnp.float32
    hi = lax.Precision.HIGHEST
    x = x_ref[...]
    cn = (cp_ref[0] + cp_ref[1])[:_N_NODES]
    giota = lax.broadcasted_iota(jnp.int32, (_N_GRAPHS, _N_NODES), 0)
    onehot_t = (b_ref[...] == giota).astype(f32)
    dn0 = (((0,), (0,)), ((), ()))
    dn1 = (((1,), (0,)), ((), ()))
    m1 = lax.dot_general(cn, x, dn0, precision=hi, preferred_element_type=f32)
    m2 = lax.dot_general(onehot_t, x, dn1, preferred_element_type=f32)
    ncol = jnp.sum(onehot_t, axis=1, keepdims=True)
    dc = (((1,), (1,)), ((), ()))
    sums = (lax.dot_general(m1, wrel_ref[...], dc, precision=hi,
                            preferred_element_type=f32)
            + lax.dot_general(m2, wroot_ref[...], dc, precision=hi,
                              preferred_element_type=f32)
            + ncol * brel_ref[...])
    pooled = sums / jnp.maximum(ncol, 1.0)
    out_ref[...] = (lax.dot_general(pooled, wlin_ref[...], dc, precision=hi,
                                    preferred_element_type=f32)
                    + blin_ref[...])


def kernel(x, edge_index, batch, W_rel, b_rel, W_root, W_lin, b_lin):
    n_classes = W_lin.shape[0]
    cpart = _edge_hist()(edge_index.reshape(-1), batch)
    cpart = cpart.reshape(_NC, _N_PAD, _N_GRAPHS)
    pool = pl.pallas_call(
        _pool_body,
        out_shape=jax.ShapeDtypeStruct((_N_GRAPHS, n_classes), jnp.float32),
    )
    return pool(x, cpart, batch.reshape(1, _N_NODES), W_rel,
                b_rel.reshape(1, -1), W_root, W_lin, b_lin.reshape(1, -1))


# speedup vs baseline: 37.2163x; 1.1883x over previous
import functools

import jax
import jax.numpy as jnp
from jax import lax
from jax.experimental import pallas as pl
from jax.experimental.pallas import tpu as pltpu
from jax.experimental.pallas import tpu_sc as plsc

_N_NODES = 10000
_N_EDGES = 320000
_N_GRAPHS = 64
_D = 128
_NC = 2
_NS = 16
_N_PAD = 10240
_KEYS = _N_PAD * _N_GRAPHS
_SLICE = _KEYS // _NS
_EPT = _N_EDGES // (_NC * _NS)
_EPTP = 10240
_EWIN = 10496
_ICH = 128
_NCH = _EPTP // _ICH
_PADKEY = _KEYS - 1


def _hist_body(edge_hbm, batch_hbm, out_hbm,
               batch_v, ebuf_v, key_v, ones_v, stage_v, c_sh, sem):
    cid = lax.axis_index("c")
    sid = lax.axis_index("s")
    t = cid * _NS + sid
    nominal = t * _EPT
    start = pl.multiple_of(
        jnp.minimum(nominal - lax.rem(nominal, 128), _N_EDGES - _EWIN), 128)
    loff = nominal - start

    pltpu.sync_copy(batch_hbm, batch_v)
    pltpu.sync_copy(edge_hbm.at[:, pl.ds(start, _EWIN)], ebuf_v)

    for i in range(0, _ICH, 16):
        ones_v[pl.ds(i, 16)] = jnp.ones((16,), jnp.float32)

    def _zero(i, carry):
        stage_v[pl.ds(i * 16, 16)] = jnp.zeros((16,), jnp.float32)
        return carry

    lax.fori_loop(0, _SLICE // 16, _zero, 0, unroll=8)
    pltpu.sync_copy(stage_v, c_sh.at[pl.ds(sid * _SLICE, _SLICE)])

    lane = lax.broadcasted_iota(jnp.int32, (16,), 0)
    padkey = jnp.full((16,), _PADKEY, jnp.int32)

    def _keys(j, carry):
        for u in range(_ICH // 16):
            e0 = j * _ICH + u * 16
            off = jnp.minimum(loff + e0, _EWIN - 16)
            sv = ebuf_v[0, pl.ds(off, 16)]
            dv = ebuf_v[1, pl.ds(off, 16)]
            ge = plsc.load_gather(batch_v, [dv])
            real = (e0 + lane) < _EPT
            key_v[j, pl.ds(u * 16, 16)] = jnp.where(
                real, sv * _N_GRAPHS + ge, padkey)
        return carry

    lax.fori_loop(0, _NCH, _keys, 0)

    plsc.subcore_barrier()

    def _fire(j, carry):
        pltpu.async_copy(ones_v, c_sh.at[key_v.at[j]], sem, add=True)
        return carry

    lax.fori_loop(0, _NCH, _fire, 0)

    def _drain(j, carry):
        pltpu.make_async_copy(ones_v, c_sh.at[key_v.at[0]], sem).wait()
        return carry

    lax.fori_loop(0, _NCH, _drain, 0)

    plsc.subcore_barrier()

    pltpu.sync_copy(c_sh.at[pl.ds(sid * _SLICE, _SLICE)], stage_v)
    pltpu.sync_copy(stage_v, out_hbm.at[pl.ds(cid * _KEYS + sid * _SLICE,
                                              _SLICE)])


@functools.cache
def _edge_hist():
    return functools.partial(
        pl.kernel,
        mesh=plsc.VectorSubcoreMesh(core_axis_name="c", subcore_axis_name="s"),
        out_type=jax.ShapeDtypeStruct((_NC * _KEYS,), jnp.float32),
        compiler_params=pltpu.CompilerParams(needs_layout_passes=False),
        scratch_types=[
            pltpu.VMEM((_N_NODES,), jnp.int32),
            pltpu.VMEM((2, _EWIN), jnp.int32),
            pltpu.VMEM((_NCH, _ICH), jnp.int32),
            pltpu.VMEM((_ICH,), jnp.float32),
            pltpu.VMEM((_SLICE,), jnp.float32),
            pltpu.VMEM_SHARED((_KEYS,), jnp.float32),
            pltpu.SemaphoreType.DMA,
        ],
    )(_hist_body)


def _pool_body(x_ref, cp_ref, b_ref, wrel_ref, brel_ref, wroot_ref,
               wlin_ref, blin_ref, out_ref):
    f32 = jnp.float32
    hi = lax.Precision.HIGHEST
    x = x_ref[...]
    csum = (cp_ref[pl.ds(0, _N_PAD // 2), :]
            + cp_ref[pl.ds(_N_PAD // 2, _N_PAD // 2), :])[:_N_NODES // 2]
    xr = x.reshape(_N_NODES // 2, 2 * _D)
    dn0 = (((0,), (0,)), ((), ()))
    dn1 = (((1,), (0,)), ((), ()))
    mm = lax.dot_general(csum, xr, dn0, precision=hi,
                         preferred_element_type=f32)
    m1 = mm[:_N_GRAPHS, :_D] + mm[_N_GRAPHS:, _D:]
    giota = lax.broadcasted_iota(jnp.int32, (_N_GRAPHS, _N_NODES), 0)
    onehot_t = (b_ref[...] == giota).astype(f32)
    m2 = lax.dot_general(onehot_t, x, dn1, preferred_element_type=f32)
    ncol = jnp.sum(onehot_t, axis=1, keepdims=True)
    dc = (((1,), (1,)), ((), ()))
    sums = (lax.dot_general(m1, wrel_ref[...], dc, precision=hi,
                            preferred_element_type=f32)
            + lax.dot_general(m2, wroot_ref[...], dc, precision=hi,
                              preferred_element_type=f32)
            + ncol * brel_ref[...])
    pooled = sums / jnp.maximum(ncol, 1.0)
    out_ref[...] = (lax.dot_general(pooled, wlin_ref[...], dc, precision=hi,
                                    preferred_element_type=f32)
                    + blin_ref[...])


def kernel(x, edge_index, batch, W_rel, b_rel, W_root, W_lin, b_lin):
    n_classes = W_lin.shape[0]
    cflat = _edge_hist()(edge_index, batch)
    cview = cflat.reshape(_N_PAD, _D)
    pool = pl.pallas_call(
        _pool_body,
        out_shape=jax.ShapeDtypeStruct((_N_GRAPHS, n_classes), jnp.float32),
    )
    return pool(x, cview, batch.reshape(1, _N_NODES), W_rel,
                b_rel.reshape(1, -1), W_root, W_lin, b_lin.reshape(1, -1))


# speedup vs baseline: 43.7069x; 1.1744x over previous
import functools

import jax
import jax.numpy as jnp
from jax import lax
from jax.experimental import pallas as pl
from jax.experimental.pallas import tpu as pltpu
from jax.experimental.pallas import tpu_sc as plsc

_N_NODES = 10000
_N_EDGES = 320000
_N_GRAPHS = 64
_D = 128
_NC = 2
_NS = 16
_N_PAD = 10240
_KEYS = _N_PAD * _N_GRAPHS
_SLICE = _KEYS // _NS
_EPT = _N_EDGES // (_NC * _NS)
_EPTP = 10240
_EWIN = 10496
_ICH = 128
_NCH = _EPTP // _ICH
_PADKEY = _KEYS - 1


def _hist_body(edge_hbm, batch_hbm, out_hbm,
               batch_v, ebuf_v, key_v, ones_v, stage_v, c_sh,
               sem, semb, seme):
    cid = lax.axis_index("c")
    sid = lax.axis_index("s")
    t = cid * _NS + sid
    nominal = t * _EPT
    start = pl.multiple_of(
        jnp.minimum(nominal - lax.rem(nominal, 128), _N_EDGES - _EWIN), 128)
    loff = nominal - start

    bcopy = pltpu.async_copy(batch_hbm, batch_v, semb)
    ecopy = pltpu.async_copy(edge_hbm.at[:, pl.ds(start, _EWIN)], ebuf_v, seme)

    for i in range(0, _ICH, 16):
        ones_v[pl.ds(i, 16)] = jnp.ones((16,), jnp.float32)

    def _zero(i, carry):
        stage_v[pl.ds(i * 16, 16)] = jnp.zeros((16,), jnp.float32)
        return carry

    lax.fori_loop(0, _SLICE // 16, _zero, 0, unroll=8)
    pltpu.sync_copy(stage_v, c_sh.at[pl.ds(sid * _SLICE, _SLICE)])
    bcopy.wait()
    ecopy.wait()
    plsc.subcore_barrier()

    lane = lax.broadcasted_iota(jnp.int32, (16,), 0)
    padkey = jnp.full((16,), _PADKEY, jnp.int32)

    def _keys(j, carry):
        for u in range(_ICH // 16):
            e0 = j * _ICH + u * 16
            off = jnp.minimum(loff + e0, _EWIN - 16)
            sv = ebuf_v[0, pl.ds(off, 16)]
            dv = ebuf_v[1, pl.ds(off, 16)]
            ge = plsc.load_gather(batch_v, [dv])
            real = (e0 + lane) < _EPT
            key_v[j, pl.ds(u * 16, 16)] = jnp.where(
                real, sv * _N_GRAPHS + ge, padkey)
        pltpu.async_copy(ones_v, c_sh.at[key_v.at[j]], sem, add=True)
        return carry

    lax.fori_loop(0, _NCH, _keys, 0)

    def _drain(j, carry):
        pltpu.make_async_copy(ones_v, c_sh.at[key_v.at[0]], sem).wait()
        return carry

    lax.fori_loop(0, _NCH, _drain, 0)

    plsc.subcore_barrier()

    pltpu.sync_copy(c_sh.at[pl.ds(sid * _SLICE, _SLICE)],
                    out_hbm.at[pl.ds(cid * _KEYS + sid * _SLICE, _SLICE)])


@functools.cache
def _edge_hist():
    return functools.partial(
        pl.kernel,
        mesh=plsc.VectorSubcoreMesh(core_axis_name="c", subcore_axis_name="s"),
        out_type=jax.ShapeDtypeStruct((_NC * _KEYS,), jnp.float32),
        compiler_params=pltpu.CompilerParams(needs_layout_passes=False),
        scratch_types=[
            pltpu.VMEM((_N_NODES,), jnp.int32),
            pltpu.VMEM((2, _EWIN), jnp.int32),
            pltpu.VMEM((_NCH, _ICH), jnp.int32),
            pltpu.VMEM((_ICH,), jnp.float32),
            pltpu.VMEM((_SLICE,), jnp.float32),
            pltpu.VMEM_SHARED((_KEYS,), jnp.float32),
            pltpu.SemaphoreType.DMA,
            pltpu.SemaphoreType.DMA,
            pltpu.SemaphoreType.DMA,
        ],
    )(_hist_body)


def _pool_body(x_ref, cp_ref, b_ref, wrel_ref, brel_ref, wroot_ref,
               wlin_ref, blin_ref, out_ref):
    f32 = jnp.float32
    hi = lax.Precision.HIGHEST
    x = x_ref[...]
    csum = (cp_ref[pl.ds(0, _N_PAD // 2), :]
            + cp_ref[pl.ds(_N_PAD // 2, _N_PAD // 2), :])[:_N_NODES // 2]
    xr = x.reshape(_N_NODES // 2, 2 * _D)
    dn0 = (((0,), (0,)), ((), ()))
    dn1 = (((1,), (0,)), ((), ()))
    mm = lax.dot_general(csum, xr, dn0, precision=hi,
                         preferred_element_type=f32)
    m1 = mm[:_N_GRAPHS, :_D] + mm[_N_GRAPHS:, _D:]
    giota = lax.broadcasted_iota(jnp.int32, (_N_GRAPHS, _N_NODES), 0)
    onehot_t = (b_ref[...] == giota).astype(f32)
    m2 = lax.dot_general(onehot_t, x, dn1, preferred_element_type=f32)
    ncol = jnp.sum(onehot_t, axis=1, keepdims=True)
    dc = (((1,), (1,)), ((), ()))
    sums = (lax.dot_general(m1, wrel_ref[...], dc, precision=hi,
                            preferred_element_type=f32)
            + lax.dot_general(m2, wroot_ref[...], dc, precision=hi,
                              preferred_element_type=f32)
            + ncol * brel_ref[...])
    pooled = sums / jnp.maximum(ncol, 1.0)
    out_ref[...] = (lax.dot_general(pooled, wlin_ref[...], dc, precision=hi,
                                    preferred_element_type=f32)
                    + blin_ref[...])


def kernel(x, edge_index, batch, W_rel, b_rel, W_root, W_lin, b_lin):
    n_classes = W_lin.shape[0]
    cflat = _edge_hist()(edge_index, batch)
    cview = cflat.reshape(_N_PAD, _D)
    pool = pl.pallas_call(
        _pool_body,
        out_shape=jax.ShapeDtypeStruct((_N_GRAPHS, n_classes), jnp.float32),
    )
    return pool(x, cview, batch.reshape(1, _N_NODES), W_rel,
                b_rel.reshape(1, -1), W_root, W_lin, b_lin.reshape(1, -1))


# speedup vs baseline: 43.7698x; 1.0014x over previous
import functools

import jax
import jax.numpy as jnp
from jax import lax
from jax.experimental import pallas as pl
from jax.experimental.pallas import tpu as pltpu
from jax.experimental.pallas import tpu_sc as plsc

_N_NODES = 10000
_N_EDGES = 320000
_N_GRAPHS = 64
_D = 128
_NC = 2
_NS = 16
_N_PAD = 10240
_KEYS = _N_PAD * _N_GRAPHS
_SLICE = _KEYS // _NS
_EPT = _N_EDGES // (_NC * _NS)
_EPTP = 10240
_EWIN = 10496
_ICH = 128
_NCH = _EPTP // _ICH
_PADKEY = _KEYS - 1


def _hist_body(edge_hbm, batch_hbm, out_hbm,
               batch_v, ebuf_v, key_v, ones_v, stage_v, c_sh,
               sem, semb, seme):
    cid = lax.axis_index("c")
    sid = lax.axis_index("s")
    t = cid * _NS + sid
    nominal = t * _EPT
    start = pl.multiple_of(
        jnp.minimum(nominal - lax.rem(nominal, 128), _N_EDGES - _EWIN), 128)
    loff = nominal - start

    bcopy = pltpu.async_copy(batch_hbm, batch_v, semb)
    ecopy = pltpu.async_copy(edge_hbm.at[:, pl.ds(start, _EWIN)], ebuf_v, seme)

    for i in range(0, _ICH, 16):
        ones_v[pl.ds(i, 16)] = jnp.ones((16,), jnp.float32)

    def _zero(i, carry):
        stage_v[pl.ds(i * 16, 16)] = jnp.zeros((16,), jnp.float32)
        return carry

    lax.fori_loop(0, _SLICE // 16, _zero, 0, unroll=8)
    pltpu.sync_copy(stage_v, c_sh.at[pl.ds(sid * _SLICE, _SLICE)])
    bcopy.wait()
    ecopy.wait()
    plsc.subcore_barrier()

    lane = lax.broadcasted_iota(jnp.int32, (16,), 0)
    padkey = jnp.full((16,), _PADKEY, jnp.int32)

    def _keys(j, carry):
        for u in range(_ICH // 16):
            e0 = j * _ICH + u * 16
            off = jnp.minimum(loff + e0, _EWIN - 16)
            sv = ebuf_v[0, pl.ds(off, 16)]
            dv = ebuf_v[1, pl.ds(off, 16)]
            ge = plsc.load_gather(batch_v, [dv])
            real = (e0 + lane) < _EPT
            key_v[j, pl.ds(u * 16, 16)] = jnp.where(
                real, sv * _N_GRAPHS + ge, padkey)
        pltpu.async_copy(ones_v, c_sh.at[key_v.at[j]], sem, add=True)
        return carry

    lax.fori_loop(0, _NCH, _keys, 0)

    def _drain(j, carry):
        pltpu.make_async_copy(ones_v, c_sh.at[key_v.at[0]], sem).wait()
        return carry

    lax.fori_loop(0, _NCH, _drain, 0)

    plsc.subcore_barrier()

    pltpu.sync_copy(c_sh.at[pl.ds(sid * _SLICE, _SLICE)],
                    out_hbm.at[pl.ds(cid * _KEYS + sid * _SLICE, _SLICE)])


@functools.cache
def _edge_hist():
    return functools.partial(
        pl.kernel,
        mesh=plsc.VectorSubcoreMesh(core_axis_name="c", subcore_axis_name="s"),
        out_type=jax.ShapeDtypeStruct((_NC * _KEYS,), jnp.float32),
        compiler_params=pltpu.CompilerParams(needs_layout_passes=False),
        scratch_types=[
            pltpu.VMEM((_N_NODES,), jnp.int32),
            pltpu.VMEM((2, _EWIN), jnp.int32),
            pltpu.VMEM((_NCH, _ICH), jnp.int32),
            pltpu.VMEM((_ICH,), jnp.float32),
            pltpu.VMEM((_SLICE,), jnp.float32),
            pltpu.VMEM_SHARED((_KEYS,), jnp.float32),
            pltpu.SemaphoreType.DMA,
            pltpu.SemaphoreType.DMA,
            pltpu.SemaphoreType.DMA,
        ],
    )(_hist_body)


def _pool_a_body(x_ref, b_ref, m2_ref, ncol_ref):
    f32 = jnp.float32
    giota = lax.broadcasted_iota(jnp.int32, (_N_GRAPHS, _N_NODES), 0)
    onehot_t = (b_ref[...] == giota).astype(f32)
    dn1 = (((1,), (0,)), ((), ()))
    m2_ref[...] = lax.dot_general(onehot_t, x_ref[...], dn1,
                                  preferred_element_type=f32)
    ncol_ref[...] = jnp.sum(onehot_t, axis=1, keepdims=True)


def _pool_b_body(x_ref, cp_ref, m2_ref, ncol_ref, wrel_ref, brel_ref,
                 wroot_ref, wlin_ref, blin_ref, out_ref):
    f32 = jnp.float32
    hi = lax.Precision.HIGHEST
    csum = (cp_ref[pl.ds(0, _N_PAD // 2), :]
            + cp_ref[pl.ds(_N_PAD // 2, _N_PAD // 2), :])[:_N_NODES // 2]
    xr = x_ref[...].reshape(_N_NODES // 2, 2 * _D)
    dn0 = (((0,), (0,)), ((), ()))
    mm = lax.dot_general(csum, xr, dn0, precision=hi,
                         preferred_element_type=f32)
    m1 = mm[:_N_GRAPHS, :_D] + mm[_N_GRAPHS:, _D:]
    ncol = ncol_ref[...]
    dc = (((1,), (1,)), ((), ()))
    sums = (lax.dot_general(m1, wrel_ref[...], dc, precision=hi,
                            preferred_element_type=f32)
            + lax.dot_general(m2_ref[...], wroot_ref[...], dc, precision=hi,
                              preferred_element_type=f32)
            + ncol * brel_ref[...])
    pooled = sums / jnp.maximum(ncol, 1.0)
    out_ref[...] = (lax.dot_general(pooled, wlin_ref[...], dc, precision=hi,
                                    preferred_element_type=f32)
                    + blin_ref[...])


def kernel(x, edge_index, batch, W_rel, b_rel, W_root, W_lin, b_lin):
    n_classes = W_lin.shape[0]
    cflat = _edge_hist()(edge_index, batch)
    cview = cflat.reshape(_N_PAD, _D)
    pool_a = pl.pallas_call(
        _pool_a_body,
        out_shape=(jax.ShapeDtypeStruct((_N_GRAPHS, _D), jnp.float32),
                   jax.ShapeDtypeStruct((_N_GRAPHS, 1), jnp.float32)),
    )
    m2, ncol = pool_a(x, batch.reshape(1, _N_NODES))
    pool_b = pl.pallas_call(
        _pool_b_body,
        out_shape=jax.ShapeDtypeStruct((_N_GRAPHS, n_classes), jnp.float32),
    )
    return pool_b(x, cview, m2, ncol, W_rel, b_rel.reshape(1, -1), W_root,
                  W_lin, b_lin.reshape(1, -1))
